# Initial kernel scaffold; baseline (speedup 1.0000x reference)
import jax
import jax.numpy as jnp
from jax.experimental import pallas as pl


def kernel(x, perm):
    raise NotImplementedError("write your pallas kernel here")


# speedup vs baseline: 4.5623x; 4.5623x over previous
import functools

import jax
import jax.numpy as jnp
from jax import lax
from jax.experimental import pallas as pl
from jax.experimental.pallas import tpu as pltpu
from jax.experimental.pallas import tpu_sc as plsc

_NC = 2
_NS = 16
_NW = _NC * _NS

_IV = 128
_SPC = 8
_CH = _IV * _SPC


def _body(n, b, xf_hbm, perm_hbm, out_hbm, y_hbm, idx_v, rows_v, sem):
    wid = lax.axis_index("s") * _NC + lax.axis_index("c")
    per_w = n // _NW
    base_r = wid * (per_w // _IV)

    def chunk(s, c):
        off_r = base_r + s * _SPC
        pltpu.sync_copy(perm_hbm.at[pl.ds(off_r, _SPC)], idx_v)
        cps = []
        for bb in range(b):
            for i in range(_SPC):
                cps.append(pltpu.async_copy(
                    xf_hbm.at[bb].at[idx_v.at[i]], rows_v.at[bb].at[i], sem))
        for cp in cps:
            cp.wait()
        for bb in range(b):
            pltpu.sync_copy(rows_v.at[bb], out_hbm.at[bb].at[pl.ds(off_r, _SPC)])
        return c

    lax.fori_loop(0, per_w // _CH, chunk, 0)

    cy = n // _NW
    for bb in range(b):
        pltpu.sync_copy(xf_hbm.at[bb].at[pl.ds(wid * cy, cy)],
                        y_hbm.at[bb].at[pl.ds(wid * cy, cy)])


@jax.jit
def _interleave(xf, perm2):
    b, n = xf.shape
    mesh = plsc.VectorSubcoreMesh(core_axis_name="c", subcore_axis_name="s")
    k = pl.kernel(
        functools.partial(_body, n, b),
        out_type=(
            jax.ShapeDtypeStruct((b, n // _IV, _IV), jnp.float32),
            jax.ShapeDtypeStruct((b, n), jnp.float32),
        ),
        mesh=mesh,
        scratch_types=[
            pltpu.VMEM((_SPC, _IV), jnp.int32),
            pltpu.VMEM((b, _SPC, _IV), jnp.float32),
            pltpu.SemaphoreType.DMA,
        ],
        compiler_params=pltpu.CompilerParams(use_tc_tiling_on_sc=False),
    )
    return k(xf, perm2)


def kernel(x, perm):
    bsz = x.shape[0]
    n = perm.shape[0]
    out, y = _interleave(x.reshape(bsz, n), perm.reshape(n // _IV, _IV))
    return (out.reshape(x.shape), y.reshape(x.shape))
